# Initial kernel scaffold; baseline (speedup 1.0000x reference)
import jax
import jax.numpy as jnp
from jax.experimental import pallas as pl


def kernel(input, cell_1_mask, cell_2_mask, cell_1_sizes, cell_2_sizes):
    raise NotImplementedError("write your pallas kernel here")


# speedup vs baseline: 5.5333x; 5.5333x over previous
import functools

import jax
import jax.numpy as jnp
from jax import lax
from jax.experimental import pallas as pl
from jax.experimental.pallas import tpu as pltpu, tpu_sc as plsc

N, C, L = 320000, 128, 10000
NW = 32
CH = 256
NCH = N // CH
RW = 16 + CH
G8 = C // 16
SPAD = 320
LP = NW * SPAD
OC = 3 * C


def _run_meta(m, sizes):
    lab2 = m.reshape(NCH, CH)
    ends = jnp.concatenate(
        [lab2[:, 1:] != lab2[:, :-1], jnp.ones((NCH, 1), bool)], axis=1)
    iot = lax.broadcasted_iota(jnp.int32, (NCH, CH), 1)
    pos = jnp.where(ends, iot, CH)
    pos_s = jnp.sort(pos, axis=1)
    nr = jnp.sum(ends, axis=1, dtype=jnp.int32)
    prev = jnp.concatenate(
        [jnp.full((NCH, 1), -1, jnp.int32), pos_s[:, :-1]], axis=1)
    cnt = jnp.where(pos_s < CH, pos_s - prev, 0).astype(jnp.int32)
    g = lax.broadcasted_iota(jnp.int32, (NCH, CH), 0) * CH \
        + jnp.where(pos_s < CH, pos_s, 0)
    m_next = jnp.concatenate([m[1:], m[-1:]])
    segend = (g == N - 1) | (m_next[g] != m[g])
    vals = jnp.where(segend & (pos_s < CH), -cnt, cnt)
    runvals = jnp.concatenate(
        [nr[:, None], jnp.zeros((NCH, 15), jnp.int32), vals], axis=1)

    starts = jnp.concatenate(
        [jnp.zeros((1,), jnp.int32), jnp.cumsum(sizes, dtype=jnp.int32)])
    lbs = [(L * w) // NW for w in range(NW + 1)]
    lb = jnp.array(lbs[:NW], jnp.int32)
    rs = starts[jnp.array(lbs[:NW])]
    re = starts[jnp.array(lbs[1:])]
    c0 = rs // CH
    fo = rs % CH
    nch = (re - 1) // CH - c0 + 1
    ends_cum = jnp.cumsum(ends, axis=1, dtype=jnp.int32)
    ro = jnp.where(fo > 0, ends_cum[c0, jnp.maximum(fo - 1, 0)], 0)
    tm = jnp.stack([c0, fo, ro, lb, nch], axis=1).astype(jnp.int32)
    tm = jnp.pad(tm, ((0, 0), (0, 16 - tm.shape[1])))
    return runvals, tm


def _init_accs():
    return ([jnp.zeros((16,), jnp.float32) for _ in range(G8)]
            + [jnp.full((16,), jnp.inf, jnp.float32) for _ in range(G8)]
            + [jnp.full((16,), -jnp.inf, jnp.float32) for _ in range(G8)])


def _sc_body(x, rv1, rv2, tmb, sz1, sz2, o1, s1, o2, s2,
             rows_v, runs_v, meta_v, stage, szv, sbuf, fsem):
    wid = lax.axis_index("s") * 2 + lax.axis_index("c")

    for mi, (rv, sz, out, souts) in enumerate(
            ((rv1, sz1, o1, s1), (rv2, sz2, o2, s2))):
        pltpu.sync_copy(tmb.at[pl.ds(mi * NW + wid, 1)], meta_v)
        mv = meta_v[0, pl.ds(0, 16)]
        c0, fo, ro, lb, nch = mv[0], mv[1], mv[2], mv[3], mv[4]

        def chunk_body(ci, carry, c0=c0, fo=fo, ro=ro, lb=lb, rv=rv, out=out):
            out_j, seg_n = carry[0], carry[1]
            accs = list(carry[2:])
            c = c0 + ci
            pltpu.sync_copy(x.at[pl.ds(c * CH, CH)], rows_v)
            pltpu.sync_copy(rv.at[pl.ds(c, 1)], runs_v)
            nr = runs_v[0, pl.ds(0, 16)][0]
            first = ci == 0
            row0 = lax.select(first, fo, jnp.int32(0))
            rlo = lax.select(first, ro, jnp.int32(0))
            ngrp = (nr + jnp.int32(15)) >> 4

            def grp_body(gi, gc, rlo=rlo, lb=lb, out=out):
                row, out_j, seg_n = gc[0], gc[1], gc[2]
                accs = list(gc[3:])
                vals = runs_v[0, pl.ds(16 + gi * 16, 16)]
                for lane in range(16):
                    v = vals[lane]
                    idx = gi * 16 + lane
                    act = idx >= rlo
                    neg = v < jnp.int32(0)
                    is_last = act & neg
                    cnt = lax.select(
                        act, lax.select(neg, -v, v), jnp.int32(0))

                    def row_body(i, a, row=row):
                        r = row + i
                        ld = [rows_v[r, pl.ds(g * 16, 16)] for g in range(G8)]
                        return ([a[g] + ld[g] for g in range(G8)]
                                + [jnp.minimum(a[G8 + g], ld[g])
                                   for g in range(G8)]
                                + [jnp.maximum(a[2 * G8 + g], ld[g])
                                   for g in range(G8)])

                    accs = lax.fori_loop(0, cnt, row_body, accs)
                    seg_n = seg_n + cnt
                    row = row + cnt

                    def do_flush(a, out_j=out_j, seg_n=seg_n, lb=lb, out=out):
                        slot = out_j & jnp.int32(3)
                        lab = lb + out_j

                        def _w(_):
                            pltpu.make_async_copy(
                                stage.at[pl.ds(slot, 1)],
                                out.at[pl.ds(lab, 1)],
                                fsem.at[slot]).wait()
                            return jnp.int32(0)

                        lax.cond(out_j >= jnp.int32(4), _w,
                                 lambda _: jnp.int32(0), jnp.int32(0))
                        nf = jnp.full((16,), seg_n, jnp.int32) \
                            .astype(jnp.float32)
                        inv = jnp.ones((16,), jnp.float32) / nf
                        for g in range(G8):
                            stage[slot, pl.ds(g * 16, 16)] = a[g] * inv
                        for g in range(G8):
                            stage[slot, pl.ds(C + g * 16, 16)] = a[G8 + g]
                        for g in range(G8):
                            stage[slot, pl.ds(2 * C + g * 16, 16)] = \
                                a[2 * G8 + g]
                        pltpu.make_async_copy(
                            stage.at[pl.ds(slot, 1)],
                            out.at[pl.ds(lab, 1)],
                            fsem.at[slot]).start()
                        return _init_accs()

                    accs = lax.cond(is_last, do_flush, lambda a: list(a),
                                    accs)
                    out_j = lax.select(is_last, out_j + 1, out_j)
                    seg_n = lax.select(is_last, jnp.int32(0), seg_n)
                return (row, out_j, seg_n, *accs)

            gfin = lax.fori_loop(0, ngrp, grp_body,
                                 (row0, out_j, seg_n, *accs))
            return gfin[1:]

        fin = lax.fori_loop(0, nch, chunk_body,
                            (jnp.int32(0), jnp.int32(0), *_init_accs()))
        out_j = fin[0]

        def drain(i, _, lb=lb, out=out):
            sl = i & jnp.int32(3)
            pltpu.make_async_copy(
                stage.at[pl.ds(sl, 1)], out.at[pl.ds(lb, 1)],
                fsem.at[sl]).wait()
            return jnp.int32(0)

        lax.fori_loop(jnp.maximum(out_j - 4, 0), out_j, drain, jnp.int32(0))

        sb = wid * SPAD
        pltpu.sync_copy(sz.at[pl.ds(sb, SPAD)], szv)
        for k in range(SPAD // 16):
            v = szv[pl.ds(k * 16, 16)].astype(jnp.float32)
            sbuf[pl.ds(k * 16, 16)] = jnp.exp(-v) - 0.5
        pltpu.sync_copy(sbuf, souts.at[pl.ds(sb, SPAD)])


def kernel(input, cell_1_mask, cell_2_mask, cell_1_sizes, cell_2_sizes):
    rv1, tm1 = _run_meta(cell_1_mask, cell_1_sizes)
    rv2, tm2 = _run_meta(cell_2_mask, cell_2_sizes)
    tmb = jnp.concatenate([tm1, tm2], axis=0)
    pad1 = jnp.ones((LP - L,), jnp.int32)
    sz1 = jnp.concatenate([cell_1_sizes.astype(jnp.int32), pad1])
    sz2 = jnp.concatenate([cell_2_sizes.astype(jnp.int32), pad1])

    mesh = plsc.VectorSubcoreMesh(core_axis_name="c", subcore_axis_name="s")
    o1, s1, o2, s2 = pl.kernel(
        _sc_body,
        out_type=[
            jax.ShapeDtypeStruct((L, OC), jnp.float32),
            jax.ShapeDtypeStruct((LP,), jnp.float32),
            jax.ShapeDtypeStruct((L, OC), jnp.float32),
            jax.ShapeDtypeStruct((LP,), jnp.float32),
        ],
        mesh=mesh,
        scratch_types=[
            pltpu.VMEM((CH, C), jnp.float32),
            pltpu.VMEM((1, RW), jnp.int32),
            pltpu.VMEM((1, 16), jnp.int32),
            pltpu.VMEM((4, OC), jnp.float32),
            pltpu.VMEM((SPAD,), jnp.int32),
            pltpu.VMEM((SPAD,), jnp.float32),
            pltpu.SemaphoreType.DMA((4,)),
        ],
    )(input, rv1, rv2, tmb, sz1, sz2)

    r1 = jnp.concatenate([o1, s1[:L, None]], axis=1)
    r2 = jnp.concatenate([o2, s2[:L, None]], axis=1)
    return (r1, r2)
